# baseline (speedup 1.0000x reference)
import functools

import jax
import jax.numpy as jnp
from jax import lax
from jax.experimental import pallas as pl
from jax.experimental.pallas import tpu as pltpu
from jax.experimental.pallas import tpu_sc as plsc

_N_FIELDS = 26
_VOCAB = 100000
_EMBED_DIM = 16
_BATCH = 16384
_HASH_BUCKETS = 1000

_TOTAL = _BATCH * _N_FIELDS
_NC = 2
_NS = 16
_NW = _NC * _NS
_BLK = 128
_BLOCKS = _TOTAL // _BLK
_BPW = 105


def _prep_body(idx_ref, h_ref, g_ref):
    x = idx_ref[...]
    xu = x.astype(jnp.uint32)
    h_ref[...] = ((xu * jnp.uint32(2654435761)) % jnp.uint32(_HASH_BUCKETS)
                  ).astype(jnp.int32)
    col = lax.broadcasted_iota(jnp.int32, (_BATCH, _N_FIELDS), 1)
    g_ref[...] = x + col * _VOCAB


def _prep_call(indices):
    return pl.pallas_call(
        _prep_body,
        out_shape=(
            jax.ShapeDtypeStruct((_BATCH, _N_FIELDS), jnp.int32),
            jax.ShapeDtypeStruct((_BATCH, _N_FIELDS), jnp.int32),
        ),
    )(indices)


_MESH = plsc.VectorSubcoreMesh(core_axis_name="c", subcore_axis_name="s")


@functools.partial(
    pl.kernel,
    mesh=_MESH,
    compiler_params=pltpu.CompilerParams(use_tc_tiling_on_sc=False),
    out_type=jax.ShapeDtypeStruct((_TOTAL, _EMBED_DIM), jnp.float32),
    scratch_types=[
        pltpu.VMEM((_BLK,), jnp.int32),
        pltpu.VMEM((_BLK, _EMBED_DIM), jnp.float32),
        pltpu.SemaphoreType.DMA,
    ],
)
def _sc_gather(idx_hbm, tab_hbm, out_hbm, idx_v, rows_v, sem):
    wid = lax.axis_index("s") * _NC + lax.axis_index("c")
    base = (wid * (_BLOCKS - _BPW)) // (_NW - 1)

    def do_blk(j, carry):
        off = (base + j) * _BLK
        pltpu.sync_copy(idx_hbm.at[pl.ds(off, _BLK)], idx_v)
        pltpu.async_copy(tab_hbm.at[idx_v], rows_v, sem).wait()
        pltpu.sync_copy(rows_v, out_hbm.at[pl.ds(off, _BLK)])
        return carry

    lax.fori_loop(0, _BPW, do_blk, 0)


def kernel(indices, tables):
    h, g = _prep_call(indices)
    flat_idx = g.reshape(_TOTAL)
    tab_flat = tables.reshape(_N_FIELDS * _VOCAB, _EMBED_DIM)
    out = _sc_gather(flat_idx, tab_flat)
    return out.reshape(_BATCH, _N_FIELDS * _EMBED_DIM), h


# speedup vs baseline: 1.0372x; 1.0372x over previous
import functools

import jax
import jax.numpy as jnp
from jax import lax
from jax.experimental import pallas as pl
from jax.experimental.pallas import tpu as pltpu
from jax.experimental.pallas import tpu_sc as plsc

_N_FIELDS = 26
_VOCAB = 100000
_EMBED_DIM = 16
_BATCH = 16384
_HASH_BUCKETS = 1000

_TOTAL = _BATCH * _N_FIELDS
_NC = 2
_NS = 16
_NW = _NC * _NS
_BLK = 128
_BLOCKS = _TOTAL // _BLK
_BPW = 105


def _prep_body(idx_ref, h_ref, g_ref):
    x = idx_ref[...]
    xu = x.astype(jnp.uint32)
    h_ref[...] = ((xu * jnp.uint32(2654435761)) % jnp.uint32(_HASH_BUCKETS)
                  ).astype(jnp.int32)
    col = lax.broadcasted_iota(jnp.int32, (_BATCH, _N_FIELDS), 1)
    g_ref[...] = x + col * _VOCAB


def _prep_call(indices):
    return pl.pallas_call(
        _prep_body,
        out_shape=(
            jax.ShapeDtypeStruct((_BATCH, _N_FIELDS), jnp.int32),
            jax.ShapeDtypeStruct((_BATCH, _N_FIELDS), jnp.int32),
        ),
    )(indices)


_MESH = plsc.VectorSubcoreMesh(core_axis_name="c", subcore_axis_name="s")


@functools.partial(
    pl.kernel,
    mesh=_MESH,
    compiler_params=pltpu.CompilerParams(use_tc_tiling_on_sc=False),
    out_type=jax.ShapeDtypeStruct((_TOTAL, _EMBED_DIM), jnp.float32),
    scratch_types=[
        pltpu.VMEM((_BPW * _BLK,), jnp.int32),
        pltpu.VMEM((_BLK, _EMBED_DIM), jnp.float32),
        pltpu.SemaphoreType.DMA,
    ],
)
def _sc_gather(idx_hbm, tab_hbm, out_hbm, idx_v, rows_v, sem):
    wid = lax.axis_index("s") * _NC + lax.axis_index("c")
    base = (wid * (_BLOCKS - _BPW)) // (_NW - 1)

    pltpu.sync_copy(idx_hbm.at[pl.ds(base * _BLK, _BPW * _BLK)], idx_v)

    def do_blk(j, carry):
        off = (base + j) * _BLK
        pltpu.async_copy(
            tab_hbm.at[idx_v.at[pl.ds(j * _BLK, _BLK)]], rows_v, sem).wait()
        pltpu.sync_copy(rows_v, out_hbm.at[pl.ds(off, _BLK)])
        return carry

    lax.fori_loop(0, _BPW, do_blk, 0)


def kernel(indices, tables):
    h, g = _prep_call(indices)
    flat_idx = g.reshape(_TOTAL)
    tab_flat = tables.reshape(_N_FIELDS * _VOCAB, _EMBED_DIM)
    out = _sc_gather(flat_idx, tab_flat)
    return out.reshape(_BATCH, _N_FIELDS * _EMBED_DIM), h


# speedup vs baseline: 1.0718x; 1.0334x over previous
import functools

import jax
import jax.numpy as jnp
from jax import lax
from jax.experimental import pallas as pl
from jax.experimental.pallas import tpu as pltpu
from jax.experimental.pallas import tpu_sc as plsc

_N_FIELDS = 26
_VOCAB = 100000
_EMBED_DIM = 16
_BATCH = 16384
_HASH_BUCKETS = 1000

_TOTAL = _BATCH * _N_FIELDS
_NC = 2
_NS = 16
_NW = _NC * _NS
_BLK = 128
_BLOCKS = _TOTAL // _BLK
_BPW = 106


def _prep_body(idx_ref, h_ref, g_ref):
    x = idx_ref[...]
    xu = x.astype(jnp.uint32)
    h_ref[...] = ((xu * jnp.uint32(2654435761)) % jnp.uint32(_HASH_BUCKETS)
                  ).astype(jnp.int32)
    col = lax.broadcasted_iota(jnp.int32, (_BATCH, _N_FIELDS), 1)
    g_ref[...] = x + col * _VOCAB


def _prep_call(indices):
    return pl.pallas_call(
        _prep_body,
        out_shape=(
            jax.ShapeDtypeStruct((_BATCH, _N_FIELDS), jnp.int32),
            jax.ShapeDtypeStruct((_BATCH, _N_FIELDS), jnp.int32),
        ),
    )(indices)


_MESH = plsc.VectorSubcoreMesh(core_axis_name="c", subcore_axis_name="s")


@functools.partial(
    pl.kernel,
    mesh=_MESH,
    compiler_params=pltpu.CompilerParams(use_tc_tiling_on_sc=False),
    out_type=jax.ShapeDtypeStruct((_TOTAL, _EMBED_DIM), jnp.float32),
    scratch_types=[
        pltpu.VMEM((_BPW * _BLK,), jnp.int32),
        pltpu.VMEM((_BLK, _EMBED_DIM), jnp.float32),
        pltpu.VMEM((_BLK, _EMBED_DIM), jnp.float32),
        pltpu.SemaphoreType.DMA,
        pltpu.SemaphoreType.DMA,
        pltpu.SemaphoreType.DMA,
        pltpu.SemaphoreType.DMA,
    ],
)
def _sc_gather(idx_hbm, tab_hbm, out_hbm, idx_v, rows_a, rows_b,
               gsem_a, gsem_b, osem_a, osem_b):
    wid = lax.axis_index("s") * _NC + lax.axis_index("c")
    base = (wid * (_BLOCKS - _BPW)) // (_NW - 1)

    pltpu.sync_copy(idx_hbm.at[pl.ds(base * _BLK, _BPW * _BLK)], idx_v)

    def do_pair(jo, carry):
        j0 = 2 * jo
        ga = pltpu.async_copy(
            tab_hbm.at[idx_v.at[pl.ds(j0 * _BLK, _BLK)]], rows_a, gsem_a)
        gb = pltpu.async_copy(
            tab_hbm.at[idx_v.at[pl.ds((j0 + 1) * _BLK, _BLK)]], rows_b, gsem_b)
        ga.wait()
        oa = pltpu.async_copy(
            rows_a, out_hbm.at[pl.ds((base + j0) * _BLK, _BLK)], osem_a)
        gb.wait()
        ob = pltpu.async_copy(
            rows_b, out_hbm.at[pl.ds((base + j0 + 1) * _BLK, _BLK)], osem_b)
        oa.wait()
        ob.wait()
        return carry

    lax.fori_loop(0, _BPW // 2, do_pair, 0)


def kernel(indices, tables):
    h, g = _prep_call(indices)
    flat_idx = g.reshape(_TOTAL)
    tab_flat = tables.reshape(_N_FIELDS * _VOCAB, _EMBED_DIM)
    out = _sc_gather(flat_idx, tab_flat)
    return out.reshape(_BATCH, _N_FIELDS * _EMBED_DIM), h
